# Initial kernel scaffold; baseline (speedup 1.0000x reference)
import jax
import jax.numpy as jnp
from jax.experimental import pallas as pl


def kernel(query, support, query_left_connections, query_left_degrees, query_right_connections, query_right_degrees, support_left_connections, support_left_degrees, support_right_connections, support_right_degrees, emb_table, gcn_W, gcn_b, fc1_W, fc1_b, fc2_W, fc2_b, p1_W, p1_b, p2_W, p2_b, ln_a, ln_b, logit_W, logit_b):
    raise NotImplementedError("write your pallas kernel here")


# speedup vs baseline: 3.1855x; 3.1855x over previous
import functools

import jax
import jax.numpy as jnp
from jax import lax
from jax.experimental import pallas as pl
from jax.experimental.pallas import tpu as pltpu
from jax.experimental.pallas import tpu_sc as plsc

D = 512
HALF = 256
L = 50
LP = 56
B = 1024
FEW = 5
NSYM = 100000
SP1 = 1024
SP2 = 4096
EPS = 1e-3
BLK = 256
NBLK = 5
ROWS = BLK * NBLK

_info = plsc.get_sparse_core_info()
NC, NS = _info.num_cores, _info.num_subcores
NW = NC * NS
NJOBS = 2 * B + 2 * FEW
JOBS_PW = -(-NJOBS // NW)
NJOBS_PAD = JOBS_PW * NW
NENT = 2 * B + 2 * FEW
ENT_PW = -(-NENT // (8 * NW)) * 8
NENT_PAD = ENT_PW * NW


def _sc_body(emb, nidx, eidx, nsum, erows, idx_v, rows_v, out_v, eidx_v,
             erows_v, sem):
    wid = lax.axis_index("s") * NC + lax.axis_index("c")
    pltpu.sync_copy(nidx.at[wid], idx_v)

    @pl.loop(0, JOBS_PW)
    def _jobs(j):
        pltpu.async_copy(emb.at[idx_v.at[j]], rows_v, sem).wait()

        def red(r, accs):
            return tuple(accs[c] + rows_v[r, pl.ds(c * 16, 16)]
                         for c in range(D // 16))

        accs = lax.fori_loop(
            0, LP, red,
            tuple(jnp.zeros((16,), jnp.float32) for _ in range(D // 16)))
        for c in range(D // 16):
            out_v[j, pl.ds(c * 16, 16)] = accs[c]

    pltpu.sync_copy(out_v, nsum.at[wid])

    pltpu.sync_copy(eidx.at[wid, 0], eidx_v)
    pltpu.async_copy(emb.at[eidx_v], erows_v, sem).wait()
    pltpu.sync_copy(erows_v, erows.at[wid])


_sc_gather = pl.kernel(
    _sc_body,
    out_type=(
        jax.ShapeDtypeStruct((NW, JOBS_PW, D), jnp.float32),
        jax.ShapeDtypeStruct((NW, ENT_PW, D), jnp.float32),
    ),
    mesh=plsc.VectorSubcoreMesh(core_axis_name="c", subcore_axis_name="s"),
    scratch_types=[
        pltpu.VMEM((JOBS_PW, LP), jnp.int32),
        pltpu.VMEM((LP, D), jnp.float32),
        pltpu.VMEM((JOBS_PW, D), jnp.float32),
        pltpu.VMEM((ENT_PW,), jnp.int32),
        pltpu.VMEM((ENT_PW, D), jnp.float32),
        pltpu.SemaphoreType.DMA,
    ],
)


def _mm(a, w):
    return lax.dot_general(a, w, (((1,), (1,)), ((), ())),
                           preferred_element_type=jnp.float32)


def _tc_body(a_l, a_r, a_1, a_2, dl, dr, gcn_W, gcn_b, fc1_W, fc1_b, fc2_W,
             fc2_b, p1_W, p1_b, p2_W, p2_b, ln_a, ln_b, logit_W, logit_b,
             g_out, m_out, lp_out, smean):
    i = pl.program_id(0)
    lf = jnp.float32(L)
    left = jnp.tanh((_mm(a_l[...], gcn_W[...]) + lf * gcn_b[...]) / dl[...])
    right = jnp.tanh((_mm(a_r[...], gcn_W[...]) + lf * gcn_b[...]) / dr[...])
    h1 = _mm(a_1[...], fc1_W[...]) + fc1_b[...]
    h2 = _mm(a_2[...], fc2_W[...]) + fc2_b[...]
    ent = jnp.tanh(jnp.concatenate([h1, h2], axis=1))
    x = jnp.concatenate([left, ent, right], axis=1)
    h = jnp.maximum(_mm(x, p1_W[...]) + p1_b[...], 0.0)
    z = _mm(h, p2_W[...]) + p2_b[...] + x
    mu = jnp.mean(z, axis=1, keepdims=True)
    zc = z - mu
    var = jnp.sum(zc * zc, axis=1, keepdims=True) / jnp.float32(SP1 - 1)
    g = zc / (jnp.sqrt(var) + EPS) * ln_a[...] + ln_b[...]
    g_out[...] = g

    @pl.when(i == 0)
    def _():
        smean[...] = jnp.mean(g[0:FEW, :], axis=0, keepdims=True)

    m_out[...] = _mm(g, smean[...])
    logits = _mm(g, logit_W[...]) + logit_b[...]
    mx = jnp.max(logits, axis=1, keepdims=True)
    lp_out[...] = logits - (
        mx + jnp.log(jnp.sum(jnp.exp(logits - mx), axis=1, keepdims=True)))


def _full(shape):
    return pl.BlockSpec(shape, lambda i: (0,) * len(shape))


_tc_dense = pl.pallas_call(
    _tc_body,
    grid=(NBLK,),
    in_specs=[
        pl.BlockSpec((BLK, D), lambda i: (i, 0)),
        pl.BlockSpec((BLK, D), lambda i: (i, 0)),
        pl.BlockSpec((BLK, D), lambda i: (i, 0)),
        pl.BlockSpec((BLK, D), lambda i: (i, 0)),
        pl.BlockSpec((BLK, 1), lambda i: (i, 0)),
        pl.BlockSpec((BLK, 1), lambda i: (i, 0)),
        _full((HALF, D)), _full((1, HALF)),
        _full((HALF, D)), _full((1, HALF)),
        _full((HALF, D)), _full((1, HALF)),
        _full((SP2, SP1)), _full((1, SP2)),
        _full((SP1, SP2)), _full((1, SP1)),
        _full((1, SP1)), _full((1, SP1)),
        _full((100, SP1)), _full((1, 100)),
    ],
    out_specs=[
        pl.BlockSpec((BLK, SP1), lambda i: (i, 0)),
        pl.BlockSpec((BLK, 1), lambda i: (i, 0)),
        pl.BlockSpec((BLK, 100), lambda i: (i, 0)),
    ],
    out_shape=[
        jax.ShapeDtypeStruct((ROWS, SP1), jnp.float32),
        jax.ShapeDtypeStruct((ROWS, 1), jnp.float32),
        jax.ShapeDtypeStruct((ROWS, 100), jnp.float32),
    ],
    scratch_shapes=[pltpu.VMEM((1, SP1), jnp.float32)],
    compiler_params=pltpu.CompilerParams(
        dimension_semantics=("arbitrary",)),
)


def kernel(query, support, query_left_connections, query_left_degrees,
           query_right_connections, query_right_degrees,
           support_left_connections, support_left_degrees,
           support_right_connections, support_right_degrees,
           emb_table, gcn_W, gcn_b, fc1_W, fc1_b, fc2_W, fc2_b,
           p1_W, p1_b, p2_W, p2_b, ln_a, ln_b, logit_W, logit_b):
    i32 = jnp.int32
    nidx = jnp.full((NJOBS_PAD, LP), NSYM, dtype=i32)
    nbr = jnp.concatenate([
        query_left_connections[:, :, 1].astype(i32),
        query_right_connections[:, :, 1].astype(i32),
        support_left_connections[:, :, 1].astype(i32),
        support_right_connections[:, :, 1].astype(i32),
    ], axis=0)
    nidx = nidx.at[:NJOBS, :L].set(nbr)
    eidx = jnp.full((NENT_PAD,), NSYM, dtype=i32)
    ent = jnp.concatenate([
        query[:, 0].astype(i32), query[:, 1].astype(i32),
        support[:, 0].astype(i32), support[:, 1].astype(i32),
    ])
    eidx = eidx.at[:NENT].set(ent)

    nsum, erows = _sc_gather(emb_table,
                             nidx.reshape(NW, JOBS_PW, LP),
                             eidx.reshape(NW, 1, ENT_PW))
    nsum = nsum.reshape(NJOBS_PAD, D)
    erows = erows.reshape(NENT_PAD, D)

    def pack(sup, q):
        return jnp.concatenate(
            [jnp.pad(sup, ((0, BLK - FEW), (0, 0))), q], axis=0)

    a_l = pack(nsum[2 * B:2 * B + FEW], nsum[:B])
    a_r = pack(nsum[2 * B + FEW:NJOBS], nsum[B:2 * B])
    a_1 = pack(erows[2 * B:2 * B + FEW], erows[:B])
    a_2 = pack(erows[2 * B + FEW:NENT], erows[B:2 * B])

    def packd(sup, q):
        return jnp.concatenate(
            [jnp.pad(sup, (0, BLK - FEW), constant_values=1.0), q])[:, None]

    dl = packd(support_left_degrees, query_left_degrees)
    dr = packd(support_right_degrees, query_right_degrees)

    g, m, lp = _tc_dense(a_l, a_r, a_1, a_2, dl, dr, gcn_W,
                         gcn_b[None, :], fc1_W, fc1_b[None, :], fc2_W,
                         fc2_b[None, :], p1_W, p1_b[None, :], p2_W,
                         p2_b[None, :], ln_a[None, :], ln_b[None, :],
                         logit_W, logit_b[None, :])
    return (g[BLK:], m[BLK:, 0], lp[BLK:])
